# baseline (speedup 1.0000x reference)
import functools

import jax
import jax.numpy as jnp
from jax import lax
from jax.experimental import pallas as pl
from jax.experimental.pallas import tpu as pltpu
from jax.experimental.pallas import tpu_sc as plsc

BETA = 0.1
BLK = 4096


def _router_body(x_ref, wt_ref, costs_ref, probs_ref, idx_ref, tpv_ref,
                 flat_ref):
    x = x_ref[...]
    wt = wt_ref[...]
    ne = wt.shape[1]
    logits = jnp.dot(x, wt, preferred_element_type=jnp.float32)
    logits = logits - BETA * costs_ref[...]
    m = jnp.max(logits, axis=-1, keepdims=True)
    e = jnp.exp(logits - m)
    s = jnp.sum(e, axis=-1, keepdims=True)
    probs = e / s
    probs_ref[...] = probs

    iota = jax.lax.broadcasted_iota(jnp.int32, probs.shape, 1)
    m1 = jnp.max(probs, axis=-1, keepdims=True)
    idx1 = jnp.min(jnp.where(probs == m1, iota, ne), axis=-1, keepdims=True)
    masked = jnp.where(iota == idx1, -1.0, probs)
    m2 = jnp.max(masked, axis=-1, keepdims=True)
    idx2 = jnp.min(jnp.where(masked == m2, iota, ne), axis=-1, keepdims=True)

    denom = m1 + m2 + 1e-8
    idx_ref[:, 0:1] = idx1
    idx_ref[:, 1:2] = idx2
    tpv_ref[:, 0:1] = m1 / denom
    tpv_ref[:, 1:2] = m2 / denom

    rows = idx1.shape[0] // 128
    flat_ref[:rows, :] = idx1.reshape(rows, 128)
    flat_ref[rows:, :] = idx2.reshape(rows, 128)


def _tc_gate(x, W, costs):
    nt, embed = x.shape
    ne = W.shape[0]
    grid = (nt // BLK,)
    probs, idx, tpv, flat = pl.pallas_call(
        _router_body,
        grid=grid,
        in_specs=[
            pl.BlockSpec((BLK, embed), lambda i: (i, 0)),
            pl.BlockSpec((embed, ne), lambda i: (0, 0)),
            pl.BlockSpec((1, ne), lambda i: (0, 0)),
        ],
        out_specs=[
            pl.BlockSpec((BLK, ne), lambda i: (i, 0)),
            pl.BlockSpec((BLK, 2), lambda i: (i, 0)),
            pl.BlockSpec((BLK, 2), lambda i: (i, 0)),
            pl.BlockSpec((2 * BLK // 128, 128), lambda i: (i, 0)),
        ],
        out_shape=[
            jax.ShapeDtypeStruct((nt, ne), jnp.float32),
            jax.ShapeDtypeStruct((nt, 2), jnp.int32),
            jax.ShapeDtypeStruct((nt, 2), jnp.float32),
            jax.ShapeDtypeStruct((2 * nt // 128, 128), jnp.int32),
        ],
    )(x, W.T, costs.reshape(1, ne))
    return probs, idx, tpv, flat


@functools.lru_cache(maxsize=None)
def _make_sc_bincount(n_idx, ne):
    info = plsc.get_sparse_core_info()
    nc, ns = 1, info.num_subcores
    nw = nc * ns
    n_per_w = n_idx // nw
    mesh = plsc.VectorSubcoreMesh(core_axis_name="c", subcore_axis_name="s",
                                  num_cores=1)

    @functools.partial(
        pl.kernel,
        mesh=mesh,
        out_type=jax.ShapeDtypeStruct((ne,), jnp.int32),
        scratch_types=[
            pltpu.VMEM((n_per_w,), jnp.int32),
            pltpu.VMEM((n_per_w,), jnp.int32),
            pltpu.VMEM((ne,), jnp.int32),
            pltpu.VMEM((ns * ne,), jnp.int32),
            pltpu.VMEM_SHARED((ns * ne,), jnp.int32),
        ],
    )
    def sc_bincount(idx_hbm, out_hbm, idx_v, ones_v, hist_v, comb_v, hist_sh):
        sid = lax.axis_index("s")
        base = sid * n_per_w

        def _fill_ones(i, carry):
            ones_v[pl.ds(pl.multiple_of(i * 16, 16), 16)] = jnp.full(
                (16,), 1, jnp.int32)
            return carry

        lax.fori_loop(0, n_per_w // 16, _fill_ones, 0)
        pltpu.sync_copy(idx_hbm.at[pl.ds(base, n_per_w)], idx_v)

        off = sid * ne

        def _shift(i, carry):
            sl = pl.ds(pl.multiple_of(i * 16, 16), 16)
            idx_v[sl] = idx_v[sl] + off
            return carry

        lax.fori_loop(0, n_per_w // 16, _shift, 0)

        @pl.when(sid == 0)
        def _zero_hist():
            for g in range(ne // 16):
                hist_v[pl.ds(g * 16, 16)] = jnp.zeros((16,), jnp.int32)
            for r in range(ns):
                pltpu.sync_copy(hist_v, hist_sh.at[pl.ds(r * ne, ne)])

        plsc.subcore_barrier()
        pltpu.sync_copy(ones_v, hist_sh.at[idx_v], add=True)
        plsc.subcore_barrier()

        @pl.when(sid == 0)
        def _combine():
            pltpu.sync_copy(hist_sh, comb_v)
            for g in range(ne // 16):
                acc = jnp.zeros((16,), jnp.int32)
                for r in range(ns):
                    acc = acc + comb_v[pl.ds(r * ne + g * 16, 16)]
                hist_v[pl.ds(g * 16, 16)] = acc
            pltpu.sync_copy(hist_v, out_hbm)

    return sc_bincount


def kernel(x, W, costs):
    ne = W.shape[0]
    probs, idx, tpv, flat = _tc_gate(x, W, costs)
    flat_idx = flat.reshape(-1)
    counts = _make_sc_bincount(flat_idx.shape[0], ne)(flat_idx)
    return (idx, tpv, probs, counts)


# speedup vs baseline: 1.0007x; 1.0007x over previous
import functools

import jax
import jax.numpy as jnp
from jax import lax
from jax.experimental import pallas as pl
from jax.experimental.pallas import tpu as pltpu
from jax.experimental.pallas import tpu_sc as plsc

BETA = 0.1
BLK = 4096


def _router_body(x_ref, wt_ref, costs_ref, probs_ref, idx_ref, tpv_ref,
                 flat_ref):
    x = x_ref[...]
    wt = wt_ref[...]
    ne = wt.shape[1]
    logits = jnp.dot(x, wt, preferred_element_type=jnp.float32)
    logits = logits - BETA * costs_ref[...]
    m = jnp.max(logits, axis=-1, keepdims=True)
    e = jnp.exp(logits - m)
    s = jnp.sum(e, axis=-1, keepdims=True)
    probs = e / s
    probs_ref[...] = probs

    iota = jax.lax.broadcasted_iota(jnp.int32, probs.shape, 1)
    m1 = jnp.max(probs, axis=-1, keepdims=True)
    idx1 = jnp.min(jnp.where(probs == m1, iota, ne), axis=-1, keepdims=True)
    masked = jnp.where(iota == idx1, -1.0, probs)
    m2 = jnp.max(masked, axis=-1, keepdims=True)
    idx2 = jnp.min(jnp.where(masked == m2, iota, ne), axis=-1, keepdims=True)

    denom = m1 + m2 + 1e-8
    idx_ref[:, 0:1] = idx1
    idx_ref[:, 1:2] = idx2
    tpv_ref[:, 0:1] = m1 / denom
    tpv_ref[:, 1:2] = m2 / denom

    rows = idx1.shape[0] // 128
    flat_ref[:rows, :] = idx1.reshape(rows, 128)
    flat_ref[rows:, :] = idx2.reshape(rows, 128)


def _tc_gate(x, W, costs):
    nt, embed = x.shape
    ne = W.shape[0]
    grid = (nt // BLK,)
    probs, idx, tpv, flat = pl.pallas_call(
        _router_body,
        grid=grid,
        in_specs=[
            pl.BlockSpec((BLK, embed), lambda i: (i, 0)),
            pl.BlockSpec((embed, ne), lambda i: (0, 0)),
            pl.BlockSpec((1, ne), lambda i: (0, 0)),
        ],
        out_specs=[
            pl.BlockSpec((BLK, ne), lambda i: (i, 0)),
            pl.BlockSpec((BLK, 2), lambda i: (i, 0)),
            pl.BlockSpec((BLK, 2), lambda i: (i, 0)),
            pl.BlockSpec((2 * BLK // 128, 128), lambda i: (i, 0)),
        ],
        out_shape=[
            jax.ShapeDtypeStruct((nt, ne), jnp.float32),
            jax.ShapeDtypeStruct((nt, 2), jnp.int32),
            jax.ShapeDtypeStruct((nt, 2), jnp.float32),
            jax.ShapeDtypeStruct((2 * nt // 128, 128), jnp.int32),
        ],
    )(x, W.T, costs.reshape(1, ne))
    return probs, idx, tpv, flat


@functools.lru_cache(maxsize=None)
def _make_sc_bincount(n_idx, ne):
    info = plsc.get_sparse_core_info()
    nc, ns = 1, info.num_subcores
    nw = nc * ns
    n_per_w = n_idx // nw
    mesh = plsc.VectorSubcoreMesh(core_axis_name="c", subcore_axis_name="s",
                                  num_cores=1)

    @functools.partial(
        pl.kernel,
        mesh=mesh,
        out_type=jax.ShapeDtypeStruct((ne,), jnp.int32),
        scratch_types=[
            pltpu.VMEM((n_per_w,), jnp.int32),
            pltpu.VMEM((n_per_w,), jnp.int32),
            pltpu.VMEM((ne,), jnp.int32),
            pltpu.VMEM_SHARED((ne,), jnp.int32),
        ],
    )
    def sc_bincount(idx_hbm, out_hbm, idx_v, ones_v, hist_v, hist_sh):
        sid = lax.axis_index("s")
        base = sid * n_per_w

        def _fill_ones(i, carry):
            ones_v[pl.ds(pl.multiple_of(i * 16, 16), 16)] = jnp.full(
                (16,), 1, jnp.int32)
            return carry

        lax.fori_loop(0, n_per_w // 16, _fill_ones, 0)
        pltpu.sync_copy(idx_hbm.at[pl.ds(base, n_per_w)], idx_v)

        @pl.when(sid == 0)
        def _zero_hist():
            for g in range(ne // 16):
                hist_v[pl.ds(g * 16, 16)] = jnp.zeros((16,), jnp.int32)
            pltpu.sync_copy(hist_v, hist_sh)

        plsc.subcore_barrier()
        pltpu.sync_copy(ones_v, hist_sh.at[idx_v], add=True)
        plsc.subcore_barrier()

        @pl.when(sid == 0)
        def _flush():
            pltpu.sync_copy(hist_sh, out_hbm)

    return sc_bincount


def kernel(x, W, costs):
    ne = W.shape[0]
    probs, idx, tpv, flat = _tc_gate(x, W, costs)
    flat_idx = flat.reshape(-1)
    counts = _make_sc_bincount(flat_idx.shape[0], ne)(flat_idx)
    return (idx, tpv, probs, counts)


# speedup vs baseline: 1.0012x; 1.0005x over previous
import functools

import jax
import jax.numpy as jnp
from jax import lax
from jax.experimental import pallas as pl
from jax.experimental.pallas import tpu as pltpu
from jax.experimental.pallas import tpu_sc as plsc

BETA = 0.1
BLK = 4096


def _router_body(x_ref, wt_ref, costs_ref, probs_ref, idx_ref, tpv_ref,
                 flat_ref):
    x = x_ref[...]
    wt = wt_ref[...]
    ne = wt.shape[1]
    logits = jnp.dot(x, wt, preferred_element_type=jnp.float32)
    logits = logits - BETA * costs_ref[...]
    m = jnp.max(logits, axis=-1, keepdims=True)
    e = jnp.exp(logits - m)
    s = jnp.sum(e, axis=-1, keepdims=True)
    probs = e / s
    probs_ref[...] = probs

    iota = jax.lax.broadcasted_iota(jnp.int32, probs.shape, 1)
    m1 = jnp.max(probs, axis=-1, keepdims=True)
    idx1 = jnp.min(jnp.where(probs == m1, iota, ne), axis=-1, keepdims=True)
    masked = jnp.where(iota == idx1, -1.0, probs)
    m2 = jnp.max(masked, axis=-1, keepdims=True)
    idx2 = jnp.min(jnp.where(masked == m2, iota, ne), axis=-1, keepdims=True)

    denom = m1 + m2 + 1e-8
    idx_ref[:, 0:1] = idx1
    idx_ref[:, 1:2] = idx2
    tpv_ref[:, 0:1] = m1 / denom
    tpv_ref[:, 1:2] = m2 / denom

    rows = idx1.shape[0] // 128
    flat_ref[:rows, :] = idx1.reshape(rows, 128)
    flat_ref[rows:, :] = idx2.reshape(rows, 128)


def _tc_gate(x, W, costs):
    nt, embed = x.shape
    ne = W.shape[0]
    grid = (nt // BLK,)
    probs, idx, tpv, flat = pl.pallas_call(
        _router_body,
        grid=grid,
        compiler_params=pltpu.CompilerParams(
            dimension_semantics=("parallel",)),
        in_specs=[
            pl.BlockSpec((BLK, embed), lambda i: (i, 0)),
            pl.BlockSpec((embed, ne), lambda i: (0, 0)),
            pl.BlockSpec((1, ne), lambda i: (0, 0)),
        ],
        out_specs=[
            pl.BlockSpec((BLK, ne), lambda i: (i, 0)),
            pl.BlockSpec((BLK, 2), lambda i: (i, 0)),
            pl.BlockSpec((BLK, 2), lambda i: (i, 0)),
            pl.BlockSpec((2 * BLK // 128, 128), lambda i: (i, 0)),
        ],
        out_shape=[
            jax.ShapeDtypeStruct((nt, ne), jnp.float32),
            jax.ShapeDtypeStruct((nt, 2), jnp.int32),
            jax.ShapeDtypeStruct((nt, 2), jnp.float32),
            jax.ShapeDtypeStruct((2 * nt // 128, 128), jnp.int32),
        ],
    )(x, W.T, costs.reshape(1, ne))
    return probs, idx, tpv, flat


@functools.lru_cache(maxsize=None)
def _make_sc_bincount(n_idx, ne):
    info = plsc.get_sparse_core_info()
    nc, ns = 1, info.num_subcores
    nw = nc * ns
    n_per_w = n_idx // nw
    mesh = plsc.VectorSubcoreMesh(core_axis_name="c", subcore_axis_name="s",
                                  num_cores=1)

    @functools.partial(
        pl.kernel,
        mesh=mesh,
        out_type=jax.ShapeDtypeStruct((ne,), jnp.int32),
        scratch_types=[
            pltpu.VMEM((n_per_w,), jnp.int32),
            pltpu.VMEM((n_per_w,), jnp.int32),
            pltpu.VMEM((ne,), jnp.int32),
            pltpu.VMEM_SHARED((ne,), jnp.int32),
        ],
    )
    def sc_bincount(idx_hbm, out_hbm, idx_v, ones_v, hist_v, hist_sh):
        sid = lax.axis_index("s")
        base = sid * n_per_w

        def _fill_ones(i, carry):
            ones_v[pl.ds(pl.multiple_of(i * 16, 16), 16)] = jnp.full(
                (16,), 1, jnp.int32)
            return carry

        lax.fori_loop(0, n_per_w // 16, _fill_ones, 0)
        pltpu.sync_copy(idx_hbm.at[pl.ds(base, n_per_w)], idx_v)

        @pl.when(sid == 0)
        def _zero_hist():
            for g in range(ne // 16):
                hist_v[pl.ds(g * 16, 16)] = jnp.zeros((16,), jnp.int32)
            pltpu.sync_copy(hist_v, hist_sh)

        plsc.subcore_barrier()
        pltpu.sync_copy(ones_v, hist_sh.at[idx_v], add=True)
        plsc.subcore_barrier()

        @pl.when(sid == 0)
        def _flush():
            pltpu.sync_copy(hist_sh, out_hbm)

    return sc_bincount


def kernel(x, W, costs):
    ne = W.shape[0]
    probs, idx, tpv, flat = _tc_gate(x, W, costs)
    flat_idx = flat.reshape(-1)
    counts = _make_sc_bincount(flat_idx.shape[0], ne)(flat_idx)
    return (idx, tpv, probs, counts)
